# baseline (speedup 1.0000x reference)
import functools

import jax
import jax.numpy as jnp
from jax import lax
from jax.experimental import pallas as pl
from jax.experimental.pallas import tpu as pltpu
from jax.experimental.pallas import tpu_sc as plsc

N = 10000
NP = 10240
E = 320000
D = 128
NC = 2
NS = 16
NW = NC * NS
SEG = NP // NS
R, C = 125, 80

_mesh = plsc.VectorSubcoreMesh(core_axis_name="c", subcore_axis_name="s")
_params = pltpu.CompilerParams(needs_layout_passes=False)


def _zero_vmem(ref):
    def z(k, _):
        ref[pl.ds(k * 16, 16)] = jnp.zeros((16,), jnp.float32)
        return 0

    lax.fori_loop(0, NP // 16, z, 0)


@functools.partial(
    pl.kernel,
    out_type=jax.ShapeDtypeStruct((2 * NW * NP,), jnp.float32),
    mesh=_mesh,
    compiler_params=_params,
    scratch_types=[
        pltpu.VMEM((R, C), jnp.int32),
        pltpu.VMEM((R, C), jnp.int32),
        pltpu.VMEM((NP,), jnp.float32),
        pltpu.VMEM((NP,), jnp.float32),
    ],
)
def _k1(es_hbm, degp_hbm, src_v, dst_v, do_v, di_v):
    cid = lax.axis_index("c")
    sid = lax.axis_index("s")
    wid = sid * NC + cid

    _zero_vmem(do_v)
    _zero_vmem(di_v)
    pltpu.sync_copy(es_hbm.at[0, wid], src_v)
    pltpu.sync_copy(es_hbm.at[1, wid], dst_v)

    ones = jnp.ones((16,), jnp.float32)

    def row(j, _):
        for k in range(C // 16):
            s16 = src_v[j, pl.ds(k * 16, 16)]
            d16 = dst_v[j, pl.ds(k * 16, 16)]
            plsc.addupdate_scatter(do_v, [s16], ones)
            plsc.addupdate_scatter(di_v, [d16], ones)
        return 0

    lax.fori_loop(0, R, row, 0)

    pltpu.sync_copy(do_v, degp_hbm.at[pl.ds(wid * NP, NP)])
    pltpu.sync_copy(di_v, degp_hbm.at[pl.ds((NW + wid) * NP, NP)])


def _k2a_body(x_ref, w_ref, xw_ref):
    xw_ref[...] = jnp.sum(x_ref[...] * w_ref[...], axis=-1)


_k2a = pl.pallas_call(
    _k2a_body,
    out_shape=jax.ShapeDtypeStruct((N,), jnp.float32),
)


def _k2b_body(xw_ref, degs_ref, h_ref, nd_ref):
    degs = degs_ref[...]
    deg_out = jnp.sum(degs[:NW], axis=0)[:N]
    deg_in = jnp.sum(degs[NW:], axis=0)[:N]
    h_ref[...] = xw_ref[...] * lax.rsqrt(jnp.maximum(deg_out, 1.0))
    nd_ref[...] = lax.rsqrt(jnp.maximum(deg_in, 1.0))


_k2b = pl.pallas_call(
    _k2b_body,
    out_shape=(
        jax.ShapeDtypeStruct((N,), jnp.float32),
        jax.ShapeDtypeStruct((N,), jnp.float32),
    ),
)


@functools.partial(
    pl.kernel,
    out_type=jax.ShapeDtypeStruct((NW * NP,), jnp.float32),
    mesh=_mesh,
    compiler_params=_params,
    scratch_types=[
        pltpu.VMEM((R, C), jnp.int32),
        pltpu.VMEM((R, C), jnp.int32),
        pltpu.VMEM((N,), jnp.float32),
        pltpu.VMEM((NP,), jnp.float32),
    ],
)
def _k3(es_hbm, h_hbm, aggp_hbm, src_v, dst_v, h_v, agg_v):
    cid = lax.axis_index("c")
    sid = lax.axis_index("s")
    wid = sid * NC + cid

    _zero_vmem(agg_v)
    pltpu.sync_copy(es_hbm.at[0, wid], src_v)
    pltpu.sync_copy(es_hbm.at[1, wid], dst_v)
    pltpu.sync_copy(h_hbm, h_v)

    def row(j, _):
        for k in range(C // 16):
            s16 = src_v[j, pl.ds(k * 16, 16)]
            d16 = dst_v[j, pl.ds(k * 16, 16)]
            v16 = plsc.load_gather(h_v, [s16])
            plsc.addupdate_scatter(agg_v, [d16], v16)
        return 0

    lax.fori_loop(0, R, row, 0)

    pltpu.sync_copy(agg_v, aggp_hbm.at[pl.ds(wid * NP, NP)])


def _k4_body(aggs_ref, nd_ref, b_ref, out_ref):
    agg = jnp.sum(aggs_ref[...], axis=0)[:N]
    out_ref[...] = jnp.maximum(agg * nd_ref[...] + b_ref[0, 0], 0.0)


_k4 = pl.pallas_call(
    _k4_body,
    out_shape=jax.ShapeDtypeStruct((N,), jnp.float32),
)


def kernel(x, edge_index, W_mat, b):
    es = edge_index.reshape(2, NW, R, C)
    wr = W_mat.reshape(1, D)

    xw = _k2a(x, wr)
    degp = _k1(es)
    h, nd = _k2b(xw, degp.reshape(2 * NW, NP))

    aggp = _k3(es, h)
    o = _k4(aggp.reshape(NW, NP), nd, b.reshape(1, 1))
    return o.reshape(N, 1)


# speedup vs baseline: 1.0568x; 1.0568x over previous
import functools

import jax
import jax.numpy as jnp
from jax import lax
from jax.experimental import pallas as pl
from jax.experimental.pallas import tpu as pltpu
from jax.experimental.pallas import tpu_sc as plsc

N = 10000
NP = 10240
E = 320000
D = 128
NC = 2
NS = 16
NW = NC * NS
EW = E // NW
SEG = NP // NS
R, C = 125, 80
P = 4

_mesh = plsc.VectorSubcoreMesh(core_axis_name="c", subcore_axis_name="s")
_params = pltpu.CompilerParams(needs_layout_passes=False)


def _zero_fill(ref, words):
    for k in range(words // 16):
        ref[pl.ds(k * 16, 16)] = jnp.zeros((16,), jnp.float32)


def _acc_seg(seg_v, tmp_v, parts, sl, n=None):
    n = len(parts) if n is None else n
    pltpu.sync_copy(parts[0].at[sl], seg_v)
    for p in range(1, n):
        pltpu.sync_copy(parts[p].at[sl], tmp_v)
        for k in range(SEG // 16):
            s = pl.ds(k * 16, 16)
            seg_v[s] = seg_v[s] + tmp_v[s]


@functools.partial(
    pl.kernel,
    out_type=[jax.ShapeDtypeStruct((NP,), jnp.float32) for _ in range(4)],
    mesh=_mesh,
    compiler_params=_params,
    scratch_types=[
        pltpu.VMEM((R, C), jnp.int32),
        pltpu.VMEM((R, C), jnp.int32),
        pltpu.VMEM((C,), jnp.float32),
        pltpu.VMEM((SEG,), jnp.float32),
        pltpu.VMEM((SEG,), jnp.float32),
    ]
    + [pltpu.VMEM_SHARED((NP,), jnp.float32) for _ in range(2 * P)]
    + [pltpu.SemaphoreType.DMA],
)
def _k1(es_hbm, do0_hbm, di0_hbm, do1_hbm, di1_hbm,
        src_v, dst_v, ones_v, seg_v, tmp_v, *rest):
    do_sp = rest[:P]
    di_sp = rest[P:2 * P]
    sem = rest[2 * P]
    cid = lax.axis_index("c")
    sid = lax.axis_index("s")
    wid = sid * NC + cid

    _zero_fill(seg_v, SEG)
    for k in range(C // 16):
        ones_v[pl.ds(k * 16, 16)] = jnp.ones((16,), jnp.float32)

    pltpu.sync_copy(es_hbm.at[0, wid], src_v)
    pltpu.sync_copy(es_hbm.at[1, wid], dst_v)
    for p in range(P):
        pltpu.sync_copy(seg_v, do_sp[p].at[pl.ds(sid * SEG, SEG)])
        pltpu.sync_copy(seg_v, di_sp[p].at[pl.ds(sid * SEG, SEG)])
    plsc.subcore_barrier()

    def quad(t, _):
        @pl.when(t > 0)
        def _():
            pltpu.make_async_copy(do0_hbm.at[pl.ds(0, SEG)], seg_v,
                                  sem).wait()

        j0 = t * P
        for p in range(P):
            pltpu.async_copy(ones_v, do_sp[p].at[src_v.at[j0 + p]], sem,
                             add=True)
            pltpu.async_copy(ones_v, di_sp[p].at[dst_v.at[j0 + p]], sem,
                             add=True)
        return 0

    NQ = R // P
    lax.fori_loop(0, NQ, quad, 0)
    pltpu.make_async_copy(do0_hbm.at[pl.ds(0, SEG)], seg_v, sem).wait()
    for j in range(NQ * P, R):
        pltpu.async_copy(ones_v, do_sp[0].at[src_v.at[j]], sem, add=True)
        pltpu.async_copy(ones_v, di_sp[0].at[dst_v.at[j]], sem, add=True)
        pltpu.make_async_copy(do0_hbm.at[pl.ds(0, 2 * C)],
                              seg_v.at[pl.ds(0, 2 * C)], sem).wait()
    plsc.subcore_barrier()

    sl = pl.ds(sid * SEG, SEG)

    @pl.when(cid == 0)
    def _():
        _acc_seg(seg_v, tmp_v, do_sp, sl)
        pltpu.sync_copy(seg_v, do0_hbm.at[sl])
        _acc_seg(seg_v, tmp_v, di_sp, sl)
        pltpu.sync_copy(seg_v, di0_hbm.at[sl])

    @pl.when(cid == 1)
    def _():
        _acc_seg(seg_v, tmp_v, do_sp, sl)
        pltpu.sync_copy(seg_v, do1_hbm.at[sl])
        _acc_seg(seg_v, tmp_v, di_sp, sl)
        pltpu.sync_copy(seg_v, di1_hbm.at[sl])


def _k2a_body(x_ref, w_ref, xw_ref):
    xw_ref[...] = jnp.sum(x_ref[...] * w_ref[...], axis=-1)


_k2a = pl.pallas_call(
    _k2a_body,
    out_shape=jax.ShapeDtypeStruct((N,), jnp.float32),
)


def _k2b_body(xw_ref, do0_ref, di0_ref, do1_ref, di1_ref, h_ref, nd_ref):
    deg_out = do0_ref[pl.ds(0, N)] + do1_ref[pl.ds(0, N)]
    deg_in = di0_ref[pl.ds(0, N)] + di1_ref[pl.ds(0, N)]
    h_ref[...] = xw_ref[...] * lax.rsqrt(jnp.maximum(deg_out, 1.0))
    nd_ref[...] = lax.rsqrt(jnp.maximum(deg_in, 1.0))


_k2b = pl.pallas_call(
    _k2b_body,
    out_shape=(
        jax.ShapeDtypeStruct((N,), jnp.float32),
        jax.ShapeDtypeStruct((N,), jnp.float32),
    ),
)


@functools.partial(
    pl.kernel,
    out_type=[jax.ShapeDtypeStruct((NP,), jnp.float32) for _ in range(2)],
    mesh=_mesh,
    compiler_params=_params,
    scratch_types=[
        pltpu.VMEM((R, C), jnp.int32),
        pltpu.VMEM((R, C), jnp.int32),
        pltpu.VMEM((R, C), jnp.float32),
        pltpu.VMEM((N,), jnp.float32),
        pltpu.VMEM((SEG,), jnp.float32),
        pltpu.VMEM((SEG,), jnp.float32),
    ]
    + [pltpu.VMEM_SHARED((NP,), jnp.float32) for _ in range(4 * P)]
    + [pltpu.SemaphoreType.DMA],
)
def _k3(es_hbm, h_hbm, a0_hbm, a1_hbm,
        src_v, dst_v, vals_v, h_v, seg_v, tmp_v, *rest):
    agg_sp = rest[:4 * P]
    sem = rest[4 * P]
    cid = lax.axis_index("c")
    sid = lax.axis_index("s")
    wid = sid * NC + cid

    _zero_fill(seg_v, SEG)
    pltpu.sync_copy(es_hbm.at[0, wid], src_v)
    pltpu.sync_copy(es_hbm.at[1, wid], dst_v)
    pltpu.sync_copy(h_hbm, h_v)
    for p in range(4 * P):
        pltpu.sync_copy(seg_v, agg_sp[p].at[pl.ds(sid * SEG, SEG)])
    plsc.subcore_barrier()

    def quad(t, _):
        @pl.when(t > 3)
        def _():
            pltpu.make_async_copy(h_hbm.at[pl.ds(0, P * C)],
                                  seg_v.at[pl.ds(0, P * C)], sem).wait()

        j0 = t * P
        grp = lax.rem(t, 4) * P
        for p in range(P):
            j = j0 + p
            for k in range(C // 16):
                idx16 = src_v[j, pl.ds(k * 16, 16)]
                vals_v[j, pl.ds(k * 16, 16)] = plsc.load_gather(h_v, [idx16])

            for g in range(4):
                @pl.when(grp == g * P)
                def _(p=p, j=j, g=g):
                    pltpu.async_copy(vals_v.at[j],
                                     agg_sp[g * P + p].at[dst_v.at[j]],
                                     sem, add=True)
        return 0

    NQ = R // P
    lax.fori_loop(0, NQ, quad, 0)
    for _ in range(4):
        pltpu.make_async_copy(h_hbm.at[pl.ds(0, P * C)],
                              seg_v.at[pl.ds(0, P * C)], sem).wait()
    for j in range(NQ * P, R):
        for k in range(C // 16):
            idx16 = src_v[j, pl.ds(k * 16, 16)]
            vals_v[j, pl.ds(k * 16, 16)] = plsc.load_gather(h_v, [idx16])
        pltpu.async_copy(vals_v.at[j], agg_sp[0].at[dst_v.at[j]], sem,
                         add=True)
        pltpu.make_async_copy(h_hbm.at[pl.ds(0, C)],
                              seg_v.at[pl.ds(0, C)], sem).wait()
    plsc.subcore_barrier()

    sl = pl.ds(sid * SEG, SEG)
    _acc_seg(seg_v, tmp_v, agg_sp, sl, 4 * P)

    @pl.when(cid == 0)
    def _():
        pltpu.sync_copy(seg_v, a0_hbm.at[sl])

    @pl.when(cid == 1)
    def _():
        pltpu.sync_copy(seg_v, a1_hbm.at[sl])


def _k4_body(a0_ref, a1_ref, nd_ref, b_ref, out_ref):
    agg = a0_ref[pl.ds(0, N)] + a1_ref[pl.ds(0, N)]
    out_ref[...] = jnp.maximum(agg * nd_ref[...] + b_ref[0, 0], 0.0)


_k4 = pl.pallas_call(
    _k4_body,
    out_shape=jax.ShapeDtypeStruct((N,), jnp.float32),
)


def kernel(x, edge_index, W_mat, b):
    es = edge_index.reshape(2, NW, R, C)
    wr = W_mat.reshape(1, D)

    xw = _k2a(x, wr)
    do0, di0, do1, di1 = _k1(es)
    h, nd = _k2b(xw, do0, di0, do1, di1)

    a0, a1 = _k3(es, h)
    o = _k4(a0, a1, nd, b.reshape(1, 1))
    return o.reshape(N, 1)


# speedup vs baseline: 1.1407x; 1.0794x over previous
import functools

import jax
import jax.numpy as jnp
from jax import lax
from jax.experimental import pallas as pl
from jax.experimental.pallas import tpu as pltpu
from jax.experimental.pallas import tpu_sc as plsc

N = 10000
NP = 10240
E = 320000
D = 128
NC = 2
NS = 16
NW = NC * NS
EW = E // NW
SEG = NP // NS
R, C = 125, 80
P = 4

_mesh = plsc.VectorSubcoreMesh(core_axis_name="c", subcore_axis_name="s")
_params = pltpu.CompilerParams(needs_layout_passes=False)


def _zero_fill(ref, words):
    for k in range(words // 16):
        ref[pl.ds(k * 16, 16)] = jnp.zeros((16,), jnp.float32)


def _acc_seg(seg_v, tmp_v, parts, sl, n=None):
    n = len(parts) if n is None else n
    pltpu.sync_copy(parts[0].at[sl], seg_v)
    for p in range(1, n):
        pltpu.sync_copy(parts[p].at[sl], tmp_v)
        for k in range(SEG // 16):
            s = pl.ds(k * 16, 16)
            seg_v[s] = seg_v[s] + tmp_v[s]


@functools.partial(
    pl.kernel,
    out_type=[jax.ShapeDtypeStruct((NP,), jnp.float32) for _ in range(4)],
    mesh=_mesh,
    compiler_params=_params,
    scratch_types=[
        pltpu.VMEM((R, C), jnp.int32),
        pltpu.VMEM((R, C), jnp.int32),
        pltpu.VMEM((C,), jnp.float32),
        pltpu.VMEM((SEG,), jnp.float32),
        pltpu.VMEM((SEG,), jnp.float32),
    ]
    + [pltpu.VMEM_SHARED((NP,), jnp.float32) for _ in range(2 * P)]
    + [pltpu.SemaphoreType.DMA],
)
def _k1(es_hbm, do0_hbm, di0_hbm, do1_hbm, di1_hbm,
        src_v, dst_v, ones_v, seg_v, tmp_v, *rest):
    do_sp = rest[:P]
    di_sp = rest[P:2 * P]
    sem = rest[2 * P]
    cid = lax.axis_index("c")
    sid = lax.axis_index("s")
    wid = sid * NC + cid

    _zero_fill(seg_v, SEG)
    for k in range(C // 16):
        ones_v[pl.ds(k * 16, 16)] = jnp.ones((16,), jnp.float32)

    pltpu.sync_copy(es_hbm.at[0, wid], src_v)
    pltpu.sync_copy(es_hbm.at[1, wid], dst_v)
    for p in range(P):
        pltpu.sync_copy(seg_v, do_sp[p].at[pl.ds(sid * SEG, SEG)])
        pltpu.sync_copy(seg_v, di_sp[p].at[pl.ds(sid * SEG, SEG)])
    plsc.subcore_barrier()

    def quad(t, _):
        @pl.when(t > 0)
        def _():
            pltpu.make_async_copy(do0_hbm.at[pl.ds(0, SEG)], seg_v,
                                  sem).wait()

        j0 = t * P
        for p in range(P):
            pltpu.async_copy(ones_v, do_sp[p].at[src_v.at[j0 + p]], sem,
                             add=True)
            pltpu.async_copy(ones_v, di_sp[p].at[dst_v.at[j0 + p]], sem,
                             add=True)
        return 0

    NQ = R // P
    lax.fori_loop(0, NQ, quad, 0)
    pltpu.make_async_copy(do0_hbm.at[pl.ds(0, SEG)], seg_v, sem).wait()
    for j in range(NQ * P, R):
        pltpu.async_copy(ones_v, do_sp[0].at[src_v.at[j]], sem, add=True)
        pltpu.async_copy(ones_v, di_sp[0].at[dst_v.at[j]], sem, add=True)
        pltpu.make_async_copy(do0_hbm.at[pl.ds(0, 2 * C)],
                              seg_v.at[pl.ds(0, 2 * C)], sem).wait()
    plsc.subcore_barrier()

    sl = pl.ds(sid * SEG, SEG)

    @pl.when(cid == 0)
    def _():
        _acc_seg(seg_v, tmp_v, do_sp, sl)
        pltpu.sync_copy(seg_v, do0_hbm.at[sl])
        _acc_seg(seg_v, tmp_v, di_sp, sl)
        pltpu.sync_copy(seg_v, di0_hbm.at[sl])

    @pl.when(cid == 1)
    def _():
        _acc_seg(seg_v, tmp_v, do_sp, sl)
        pltpu.sync_copy(seg_v, do1_hbm.at[sl])
        _acc_seg(seg_v, tmp_v, di_sp, sl)
        pltpu.sync_copy(seg_v, di1_hbm.at[sl])


def _k2a_body(x_ref, w_ref, xw_ref):
    xw_ref[...] = jnp.sum(x_ref[...] * w_ref[...], axis=-1)


_k2a = pl.pallas_call(
    _k2a_body,
    out_shape=jax.ShapeDtypeStruct((N,), jnp.float32),
)


def _k2b_body(xw_ref, do0_ref, di0_ref, do1_ref, di1_ref, h_ref, nd_ref):
    deg_out = do0_ref[pl.ds(0, N)] + do1_ref[pl.ds(0, N)]
    deg_in = di0_ref[pl.ds(0, N)] + di1_ref[pl.ds(0, N)]
    h_ref[...] = xw_ref[...] * lax.rsqrt(jnp.maximum(deg_out, 1.0))
    nd_ref[...] = lax.rsqrt(jnp.maximum(deg_in, 1.0))


_k2b = pl.pallas_call(
    _k2b_body,
    out_shape=(
        jax.ShapeDtypeStruct((N,), jnp.float32),
        jax.ShapeDtypeStruct((N,), jnp.float32),
    ),
)


@functools.partial(
    pl.kernel,
    out_type=[jax.ShapeDtypeStruct((NP,), jnp.float32) for _ in range(2)],
    mesh=_mesh,
    compiler_params=_params,
    scratch_types=[
        pltpu.VMEM((R, C), jnp.int32),
        pltpu.VMEM((R, C), jnp.int32),
        pltpu.VMEM((R, C), jnp.float32),
        pltpu.VMEM((N,), jnp.float32),
        pltpu.VMEM((SEG,), jnp.float32),
        pltpu.VMEM((SEG,), jnp.float32),
    ]
    + [pltpu.VMEM_SHARED((NP,), jnp.float32) for _ in range(2 * P)]
    + [pltpu.SemaphoreType.DMA],
)
def _k3(es_hbm, h_hbm, a0_hbm, a1_hbm,
        src_v, dst_v, vals_v, h_v, seg_v, tmp_v, *rest):
    agg_sp = rest[:2 * P]
    sem = rest[2 * P]
    cid = lax.axis_index("c")
    sid = lax.axis_index("s")
    wid = sid * NC + cid

    _zero_fill(seg_v, SEG)
    pltpu.sync_copy(es_hbm.at[0, wid], src_v)
    pltpu.sync_copy(es_hbm.at[1, wid], dst_v)
    pltpu.sync_copy(h_hbm, h_v)
    for p in range(2 * P):
        pltpu.sync_copy(seg_v, agg_sp[p].at[pl.ds(sid * SEG, SEG)])
    plsc.subcore_barrier()

    def quad(t, _):
        @pl.when(t > 1)
        def _():
            pltpu.make_async_copy(h_hbm.at[pl.ds(0, P * C)],
                                  seg_v.at[pl.ds(0, P * C)], sem).wait()

        j0 = t * P
        grp = lax.rem(t, 2) * P
        for p in range(P):
            j = j0 + p
            for k in range(C // 16):
                idx16 = src_v[j, pl.ds(k * 16, 16)]
                vals_v[j, pl.ds(k * 16, 16)] = plsc.load_gather(h_v, [idx16])

            @pl.when(grp == 0)
            def _(p=p, j=j):
                pltpu.async_copy(vals_v.at[j], agg_sp[p].at[dst_v.at[j]],
                                 sem, add=True)

            @pl.when(grp == P)
            def _(p=p, j=j):
                pltpu.async_copy(vals_v.at[j], agg_sp[P + p].at[dst_v.at[j]],
                                 sem, add=True)
        return 0

    NQ = R // P
    lax.fori_loop(0, NQ, quad, 0)
    pltpu.make_async_copy(h_hbm.at[pl.ds(0, 2 * P * C)],
                          seg_v.at[pl.ds(0, 2 * P * C)], sem).wait()
    for j in range(NQ * P, R):
        for k in range(C // 16):
            idx16 = src_v[j, pl.ds(k * 16, 16)]
            vals_v[j, pl.ds(k * 16, 16)] = plsc.load_gather(h_v, [idx16])
        pltpu.async_copy(vals_v.at[j], agg_sp[0].at[dst_v.at[j]], sem,
                         add=True)
        pltpu.make_async_copy(h_hbm.at[pl.ds(0, C)],
                              seg_v.at[pl.ds(0, C)], sem).wait()
    plsc.subcore_barrier()

    sl = pl.ds(sid * SEG, SEG)
    _acc_seg(seg_v, tmp_v, agg_sp, sl, 2 * P)

    @pl.when(cid == 0)
    def _():
        pltpu.sync_copy(seg_v, a0_hbm.at[sl])

    @pl.when(cid == 1)
    def _():
        pltpu.sync_copy(seg_v, a1_hbm.at[sl])


def _k4_body(a0_ref, a1_ref, nd_ref, b_ref, out_ref):
    agg = a0_ref[pl.ds(0, N)] + a1_ref[pl.ds(0, N)]
    out_ref[...] = jnp.maximum(agg * nd_ref[...] + b_ref[0, 0], 0.0)


_k4 = pl.pallas_call(
    _k4_body,
    out_shape=jax.ShapeDtypeStruct((N,), jnp.float32),
)


def kernel(x, edge_index, W_mat, b):
    es = edge_index.reshape(2, NW, R, C)
    wr = W_mat.reshape(1, D)

    xw = _k2a(x, wr)
    do0, di0, do1, di1 = _k1(es)
    h, nd = _k2b(xw, do0, di0, do1, di1)

    a0, a1 = _k3(es, h)
    o = _k4(a0, a1, nd, b.reshape(1, 1))
    return o.reshape(N, 1)
